# Initial kernel scaffold; baseline (speedup 1.0000x reference)
import functools
import math

import jax
import jax.numpy as jnp
from jax import lax
from jax.experimental import pallas as pl
from jax.experimental.pallas import tpu as pltpu
from jax.experimental.pallas import tpu_sc as plsc

D_MODEL = 32
SCALE = float(math.sqrt(D_MODEL))
LANES = 16


@functools.lru_cache(maxsize=None)
def _build(B, D):
    NW = 32
    G = 128
    BPW = B // NW
    NG = BPW // G
    assert BPW * NW == B and NG * G == BPW

    mesh = plsc.VectorSubcoreMesh(core_axis_name="c", subcore_axis_name="s")

    @functools.partial(
        pl.kernel,
        mesh=mesh,
        out_type=jax.ShapeDtypeStruct((NW, NG, G, D), jnp.float32),
        scratch_types=[
            pltpu.VMEM((NG, G), jnp.int32),
            pltpu.VMEM((G, D), jnp.float32),
            pltpu.SemaphoreType.DMA,
        ],
    )
    def emb(x_hbm, table_hbm, out_hbm, idx_v, rows_v, gsem):
        wid = lax.axis_index("s") * 2 + lax.axis_index("c")
        pltpu.sync_copy(x_hbm.at[wid], idx_v)

        def group(g, carry):
            pltpu.async_copy(table_hbm.at[idx_v.at[g]], rows_v, gsem).wait()

            def scale(i, c):
                rows_v[i, pl.ds(0, LANES)] = rows_v[i, pl.ds(0, LANES)] * SCALE
                rows_v[i, pl.ds(LANES, LANES)] = (
                    rows_v[i, pl.ds(LANES, LANES)] * SCALE
                )
                return c

            lax.fori_loop(0, G, scale, 0)
            pltpu.sync_copy(rows_v, out_hbm.at[wid, g])
            return carry

        lax.fori_loop(0, NG, group, 0)

    return emb


def kernel(x, table):
    n, s = x.shape
    B = n * s
    D = table.shape[1]
    x_flat = x.reshape(-1).astype(jnp.int32)
    NW = 32
    G = 128
    x3 = x_flat.reshape(NW, B // (NW * G), G)
    out = _build(B, D)(x3, table)
    return out.reshape(n, s, D)


# speedup vs baseline: 1.1718x; 1.1718x over previous
import functools
import math

import jax
import jax.numpy as jnp
from jax import lax
from jax.experimental import pallas as pl
from jax.experimental.pallas import tpu as pltpu
from jax.experimental.pallas import tpu_sc as plsc

D_MODEL = 32
SCALE = float(math.sqrt(D_MODEL))
LANES = 16


@functools.lru_cache(maxsize=None)
def _build(B, D):
    NW = 32
    G = 128
    BPW = B // NW
    NG = BPW // G
    assert BPW * NW == B and NG * G == BPW

    mesh = plsc.VectorSubcoreMesh(core_axis_name="c", subcore_axis_name="s")

    @functools.partial(
        pl.kernel,
        mesh=mesh,
        out_type=jax.ShapeDtypeStruct((NW, NG, G, D), jnp.float32),
        scratch_types=[
            pltpu.VMEM((NG, G), jnp.int32),
            pltpu.VMEM((G, D), jnp.float32),
            pltpu.SemaphoreType.DMA,
        ],
        compiler_params=pltpu.CompilerParams(use_tc_tiling_on_sc=False),
    )
    def emb(x_hbm, table_hbm, out_hbm, idx_v, rows_v, gsem):
        wid = lax.axis_index("s") * 2 + lax.axis_index("c")
        pltpu.sync_copy(x_hbm.at[wid], idx_v)

        def group(g, carry):
            pltpu.async_copy(table_hbm.at[idx_v.at[g]], rows_v, gsem).wait()

            def scale(i, c):
                rows_v[i, pl.ds(0, LANES)] = rows_v[i, pl.ds(0, LANES)] * SCALE
                rows_v[i, pl.ds(LANES, LANES)] = (
                    rows_v[i, pl.ds(LANES, LANES)] * SCALE
                )
                return c

            lax.fori_loop(0, G, scale, 0)
            pltpu.sync_copy(rows_v, out_hbm.at[wid, g])
            return carry

        lax.fori_loop(0, NG, group, 0)

    return emb


def kernel(x, table):
    n, s = x.shape
    B = n * s
    D = table.shape[1]
    x_flat = x.reshape(-1).astype(jnp.int32)
    NW = 32
    G = 128
    x3 = x_flat.reshape(NW, B // (NW * G), G)
    out = _build(B, D)(x3, table)
    return out.reshape(n, s, D)


# speedup vs baseline: 1.4719x; 1.2561x over previous
import functools
import math

import jax
import jax.numpy as jnp
from jax import lax
from jax.experimental import pallas as pl
from jax.experimental.pallas import tpu as pltpu
from jax.experimental.pallas import tpu_sc as plsc

D_MODEL = 32
SCALE = float(math.sqrt(D_MODEL))
LANES = 16


@functools.lru_cache(maxsize=None)
def _build(B, D):
    NW = 32
    G = 128
    K = 4
    SUP = K * G
    U = 4
    BPW = B // NW
    NG = BPW // G
    NS = BPW // SUP
    assert BPW * NW == B and NS * SUP == BPW and NS % 2 == 0

    mesh = plsc.VectorSubcoreMesh(core_axis_name="c", subcore_axis_name="s")

    @functools.partial(
        pl.kernel,
        mesh=mesh,
        out_type=jax.ShapeDtypeStruct((NW, NS, SUP, D), jnp.float32),
        scratch_types=[
            pltpu.VMEM((NG, G), jnp.int32),
            pltpu.VMEM((2, SUP, D), jnp.float32),
            pltpu.SemaphoreType.DMA,
            pltpu.SemaphoreType.DMA,
        ],
        compiler_params=pltpu.CompilerParams(use_tc_tiling_on_sc=False),
    )
    def emb(x_hbm, table_hbm, out_hbm, idx_v, rows_v, gsem0, gsem1):
        gsem = (gsem0, gsem1)
        wid = lax.axis_index("s") * 2 + lax.axis_index("c")
        pltpu.sync_copy(x_hbm.at[wid], idx_v)

        def fire(sg, b):
            for j in range(K):
                pltpu.async_copy(
                    table_hbm.at[idx_v.at[sg * K + j]],
                    rows_v.at[b, pl.ds(j * G, G)],
                    gsem[b],
                )

        def drain(b):
            for j in range(K):
                pltpu.make_async_copy(
                    table_hbm.at[idx_v.at[0]],
                    rows_v.at[b, pl.ds(j * G, G)],
                    gsem[b],
                ).wait()

        def scale(b):
            def body(i, c):
                base = i * U
                for r in range(U):
                    for h in range(2):
                        sl = pl.ds(h * LANES, LANES)
                        rows_v[b, base + r, sl] = rows_v[b, base + r, sl] * SCALE
                return c

            lax.fori_loop(0, SUP // U, body, 0)

        fire(0, 0)

        def body(h, carry):
            for b in range(2):
                sg = 2 * h + b
                nb = 1 - b

                @pl.when(sg + 1 < NS)
                def _():
                    fire(sg + 1, nb)

                drain(b)
                scale(b)
                pltpu.sync_copy(rows_v.at[b], out_hbm.at[wid, sg])
            return carry

        lax.fori_loop(0, NS // 2, body, 0)

    return emb


def kernel(x, table):
    n, s = x.shape
    B = n * s
    D = table.shape[1]
    x_flat = x.reshape(-1).astype(jnp.int32)
    NW = 32
    G = 128
    x3 = x_flat.reshape(NW, B // (NW * G), G)
    out = _build(B, D)(x3, table)
    return out.reshape(n, s, D)
